# Initial kernel scaffold; baseline (speedup 1.0000x reference)
import jax
import jax.numpy as jnp
from jax.experimental import pallas as pl


def kernel(x, pos_table):
    raise NotImplementedError("write your pallas kernel here")


# speedup vs baseline: 1.4895x; 1.4895x over previous
import jax
import jax.numpy as jnp
from jax.experimental import pallas as pl

_BLOCK_S = 512


def _add_pe_kernel(x_ref, pe_ref, o_ref):
    o_ref[...] = x_ref[...] + pe_ref[...][None, :, :]


def kernel(x, pos_table):
    batch, seq_len, d_model = x.shape
    block_s = _BLOCK_S if seq_len % _BLOCK_S == 0 else seq_len
    grid = (seq_len // block_s, batch)
    return pl.pallas_call(
        _add_pe_kernel,
        grid=grid,
        in_specs=[
            pl.BlockSpec((1, block_s, d_model), lambda s, b: (b, s, 0)),
            pl.BlockSpec((block_s, d_model), lambda s, b: (s, 0)),
        ],
        out_specs=pl.BlockSpec((1, block_s, d_model), lambda s, b: (b, s, 0)),
        out_shape=jax.ShapeDtypeStruct(x.shape, x.dtype),
    )(x, pos_table[:seq_len])


# speedup vs baseline: 1.7256x; 1.1585x over previous
import jax
import jax.numpy as jnp
from jax.experimental import pallas as pl

_BLOCK_S = 512


def _add_pe_kernel(x_ref, pe_ref, o_ref):
    o_ref[...] = x_ref[...] + pe_ref[...][None, :, :]


def kernel(x, pos_table):
    batch, seq_len, d_model = x.shape
    block_s = _BLOCK_S if seq_len % _BLOCK_S == 0 else seq_len
    grid = (seq_len // block_s,)
    return pl.pallas_call(
        _add_pe_kernel,
        grid=grid,
        in_specs=[
            pl.BlockSpec((batch, block_s, d_model), lambda s: (0, s, 0)),
            pl.BlockSpec((block_s, d_model), lambda s: (s, 0)),
        ],
        out_specs=pl.BlockSpec((batch, block_s, d_model), lambda s: (0, s, 0)),
        out_shape=jax.ShapeDtypeStruct(x.shape, x.dtype),
    )(x, pos_table[:seq_len])
